# baseline (speedup 1.0000x reference)
import dataclasses
import functools

import jax
import jax.numpy as jnp
from jax import lax
from jax.experimental import pallas as pl
from jax.experimental.pallas import tpu as pltpu
from jax.experimental.pallas import tpu_sc as plsc

N = 10000
E = 320000
D = 128

NC = 2
NS = 16
NW = NC * NS

K = 128
NB = 2
CPT = 80
CHUNKS = NW * CPT
EP = CHUNKS * K
NP = 10240
RPT = NP // NS
RB = 128
GB = NP // RB

_mesh = plsc.VectorSubcoreMesh(
    core_axis_name="c", subcore_axis_name="s", num_cores=NC, num_subcores=NS
)

_cp = pltpu.CompilerParams()
if "needs_layout_passes" in pltpu.CompilerParams.__dataclass_fields__:
    _cp = dataclasses.replace(_cp, needs_layout_passes=False)


@functools.partial(
    pl.kernel,
    out_type=jax.ShapeDtypeStruct((NW, NP), jnp.float32),
    mesh=_mesh,
    compiler_params=_cp,
    scratch_types=[
        pltpu.VMEM((CPT, 2, K), jnp.int32),
        pltpu.VMEM((NP,), jnp.float32),
    ],
)
def _sc_degree(sd_hbm, out_hbm, sd_v, hist_v):
    c = lax.axis_index("c")
    s = lax.axis_index("s")
    wid = c * NS + s
    pltpu.sync_copy(sd_hbm.at[pl.ds(wid * CPT, CPT)], sd_v)

    zeros16 = jnp.zeros((16,), jnp.float32)

    @pl.loop(0, NP // 16)
    def _(i):
        hist_v[pl.ds(i * 16, 16)] = zeros16

    ones16 = jnp.ones((16,), jnp.float32)

    @pl.loop(0, CPT)
    def _(j):
        @pl.loop(0, K // 16)
        def _(k):
            idx = sd_v[j, 1, pl.ds(k * 16, 16)]
            plsc.addupdate_scatter(hist_v, [idx], ones16)

    pltpu.sync_copy(hist_v, out_hbm.at[wid])


@functools.partial(
    pl.kernel,
    out_type=jax.ShapeDtypeStruct((NC, NP, D), jnp.float32),
    mesh=_mesh,
    scratch_types=[
        pltpu.VMEM((2 * NB, K), jnp.int32),
        pltpu.VMEM((NB, K, D), jnp.float32),
        pltpu.VMEM_SHARED((NP, D), jnp.float32),
        pltpu.SemaphoreType.DMA,
        pltpu.SemaphoreType.DMA,
        pltpu.SemaphoreType.DMA,
    ],
)
def _sc_aggregate(g_hbm, sd_hbm, zeros_hbm, out_hbm, sd_v, rows_v, acc_sp,
                  semi, semg0, semg1):
    c = lax.axis_index("c")
    s = lax.axis_index("s")
    wid = c * NS + s
    semg = [semg0, semg1]
    pltpu.sync_copy(
        zeros_hbm.at[pl.ds(s * RPT, RPT)], acc_sp.at[pl.ds(s * RPT, RPT)]
    )
    plsc.subcore_barrier()

    @pl.loop(0, CPT // NB)
    def _(t):
        base = wid * CPT + t * NB
        idx_d = [
            pltpu.async_copy(
                sd_hbm.at[base + i], sd_v.at[pl.ds(2 * i, 2)], semi
            )
            for i in range(NB)
        ]
        for d in idx_d:
            d.wait()
        g_d = [
            pltpu.async_copy(
                g_hbm.at[sd_v.at[2 * i]], rows_v.at[i], semg[i]
            )
            for i in range(NB)
        ]
        for i in range(NB):
            g_d[i].wait()
            pltpu.sync_copy(
                rows_v.at[i], acc_sp.at[sd_v.at[2 * i + 1]], add=True
            )

    plsc.subcore_barrier()
    pltpu.sync_copy(
        acc_sp.at[pl.ds(s * RPT, RPT)], out_hbm.at[c].at[pl.ds(s * RPT, RPT)]
    )


def _dis_block(degp):
    indeg = jnp.sum(degp, axis=0).reshape(1, RB)
    return lax.rsqrt(1.0 + jnp.transpose(indeg))


def _pre_body(x_ref, degp_ref, w_ref, out_ref):
    dis = _dis_block(degp_ref[...])
    h = jnp.dot(x_ref[...], w_ref[...], preferred_element_type=jnp.float32)
    out_ref[...] = h * dis


def _mid_body(acc_ref, g_ref, degp_ref, w_ref, b_ref, out_ref):
    dis = _dis_block(degp_ref[...])
    sagg = acc_ref[0] + acc_ref[1] + g_ref[...]
    t = jnp.maximum(dis * sagg + b_ref[...], 0.0)
    out_ref[...] = jnp.dot(t, w_ref[...], preferred_element_type=jnp.float32) * dis


def _out_body(acc_ref, g_ref, degp_ref, wo_ref, b_ref, bo_ref, out_ref):
    dis = _dis_block(degp_ref[...])
    sagg = acc_ref[0] + acc_ref[1] + g_ref[...]
    t = jnp.maximum(dis * sagg + b_ref[...], 0.0)
    z = jnp.dot(t, wo_ref[...], preferred_element_type=jnp.float32) + bo_ref[...]
    out_ref[...] = jax.nn.sigmoid(z)


_row_spec = pl.BlockSpec((RB, D), lambda i: (i, 0))
_acc_spec = pl.BlockSpec((NC, RB, D), lambda i: (0, i, 0))
_degp_spec = pl.BlockSpec((NW, RB), lambda i: (0, i))
_w_spec = pl.BlockSpec((D, D), lambda i: (0, 0))
_b_spec = pl.BlockSpec((1, D), lambda i: (0, 0))

_tc_pre = pl.pallas_call(
    _pre_body,
    grid=(GB,),
    in_specs=[_row_spec, _degp_spec, _w_spec],
    out_specs=_row_spec,
    out_shape=jax.ShapeDtypeStruct((NP, D), jnp.float32),
)

_tc_mid = pl.pallas_call(
    _mid_body,
    grid=(GB,),
    in_specs=[_acc_spec, _row_spec, _degp_spec, _w_spec, _b_spec],
    out_specs=_row_spec,
    out_shape=jax.ShapeDtypeStruct((NP, D), jnp.float32),
)

_tc_out = pl.pallas_call(
    _out_body,
    grid=(GB,),
    in_specs=[
        _acc_spec,
        _row_spec,
        _degp_spec,
        pl.BlockSpec((D, 1), lambda i: (0, 0)),
        _b_spec,
        pl.BlockSpec((1, 1), lambda i: (0, 0)),
    ],
    out_specs=pl.BlockSpec((RB, 1), lambda i: (i, 0)),
    out_shape=jax.ShapeDtypeStruct((NP, 1), jnp.float32),
)


@jax.jit
def kernel(x, edge_index, W1, b1, W2, b2, W3, b3, Wo, bo):
    ei = jnp.pad(edge_index, ((0, 0), (0, EP - E)), constant_values=N)
    sd = ei.reshape(2, CHUNKS, K).transpose(1, 0, 2)
    xp = jnp.pad(x, ((0, NP - N), (0, 0)))
    zeros_d = jnp.zeros((NP, D), jnp.float32)

    degp = _sc_degree(sd)
    g1 = _tc_pre(xp, degp, W1)
    a1 = _sc_aggregate(g1, sd, zeros_d)
    g2 = _tc_mid(a1, g1, degp, W2, b1.reshape(1, D))
    a2 = _sc_aggregate(g2, sd, zeros_d)
    g3 = _tc_mid(a2, g2, degp, W3, b2.reshape(1, D))
    a3 = _sc_aggregate(g3, sd, zeros_d)
    y = _tc_out(a3, g3, degp, Wo, b3.reshape(1, D), bo.reshape(1, 1))
    return y[:N]
